# Initial kernel scaffold; baseline (speedup 1.0000x reference)
import jax
import jax.numpy as jnp
from jax.experimental import pallas as pl


def kernel(x_nchw, w0, b0, w1, b1, bn0_g, bn0_b, bn1_g, bn1_b):
    raise NotImplementedError("write your pallas kernel here")


# speedup vs baseline: 6.0136x; 6.0136x over previous
import functools

import jax
import jax.numpy as jnp
from jax import lax
from jax.experimental import pallas as pl
from jax.experimental.pallas import tpu as pltpu

BN_EPS = 1e-5
LEAKY_SLOPE = 0.3
KPAD = 128
CPAD = 16
PAD = 128
_VMEM_LIMIT = 48 * 1024 * 1024


def _stats_kernel(x_ref, acc_ref, *, nchunk):
    i = pl.program_id(0)

    @pl.when(i == 0)
    def _():
        acc_ref[...] = jnp.zeros_like(acc_ref)

    x = jnp.sum(x_ref[...], axis=0)
    xx = jnp.sum(x_ref[...] * x_ref[...], axis=0)
    C = x.shape[0]
    for k in range(nchunk):
        acc_ref[0:C, :] += x[:, k * 128:(k + 1) * 128]
        acc_ref[CPAD:CPAD + C, :] += xx[:, k * 128:(k + 1) * 128]


def _channel_stats(x3):
    N, C, HW = x3.shape
    B = 8 if N % 8 == 0 else 1
    acc = pl.pallas_call(
        functools.partial(_stats_kernel, nchunk=HW // 128),
        grid=(N // B,),
        in_specs=[pl.BlockSpec((B, C, HW), lambda i: (i, 0, 0))],
        out_specs=pl.BlockSpec((2 * CPAD, 128), lambda i: (0, 0)),
        out_shape=jax.ShapeDtypeStruct((2 * CPAD, 128), jnp.float32),
        compiler_params=pltpu.CompilerParams(
            dimension_semantics=("arbitrary",),
            vmem_limit_bytes=_VMEM_LIMIT),
    )(x3)
    count = float(N * HW)
    s = jnp.sum(acc[0:C, :], axis=1)
    ss = jnp.sum(acc[CPAD:CPAD + C, :], axis=1)
    mean = s / count
    var = jnp.maximum(ss / count - mean * mean, 0.0)
    inv_std = 1.0 / jnp.sqrt(var + BN_EPS)
    return mean, inv_std


def _make_aux(mean, inv_std, gamma, beta, bias):
    c = mean.shape[0]
    fo = bias.shape[1]
    scale = gamma.reshape(c) * inv_std
    shift = beta.reshape(c) - mean * scale
    aux = jnp.zeros((CPAD, 8), jnp.float32)
    aux = aux.at[:c, 0].set(scale).at[:c, 1].set(shift)
    return aux.at[:fo, 2].set(bias.reshape(fo))


def _fill_patch_t(a, apad_ref, patch_ref, W):
    C, HW = a.shape
    apad_ref[:, PAD:PAD + HW] = a
    col = lax.broadcasted_iota(jnp.int32, (C, HW), 1) % W
    mask_l = col > 0
    mask_r = col < (W - 1)
    t = 0
    for dy in range(3):
        for dx in range(3):
            off = (dy - 1) * W + (dx - 1)
            src = apad_ref[:, PAD + off:PAD + off + HW]
            if dx == 0:
                src = jnp.where(mask_l, src, 0.0)
            elif dx == 2:
                src = jnp.where(mask_r, src, 0.0)
            patch_ref[t * C:(t + 1) * C, :] = src
            t += 1


def _leaky_bn(x, aux_ref, c):
    scale = aux_ref[0:c, 0:1]
    shift = aux_ref[0:c, 1:2]
    a = x * scale + shift
    return jnp.maximum(a, LEAKY_SLOPE * a)


def _conv0_kernel(aux_ref, w_ref, x_ref, h_ref, acc_ref, apad_ref, patch_ref,
                  *, W, nchunk):
    i = pl.program_id(0)

    @pl.when(i == 0)
    def _():
        acc_ref[...] = jnp.zeros_like(acc_ref)
        apad_ref[...] = jnp.zeros_like(apad_ref)
        patch_ref[...] = jnp.zeros_like(patch_ref)

    C = x_ref.shape[1]
    a = _leaky_bn(x_ref[0], aux_ref, C)
    _fill_patch_t(a, apad_ref, patch_ref, W)
    h = jnp.dot(w_ref[...], patch_ref[...],
                preferred_element_type=jnp.float32)
    h = h + aux_ref[:, 2:3]
    h_ref[0] = h
    for k in range(nchunk):
        hk = h[:, k * 128:(k + 1) * 128]
        acc_ref[0:CPAD, :] += hk
        acc_ref[CPAD:, :] += hk * hk


def _conv0(x3, aux0, w0t, W):
    N, C, HW = x3.shape
    h, acc = pl.pallas_call(
        functools.partial(_conv0_kernel, W=W, nchunk=HW // 128),
        grid=(N,),
        in_specs=[
            pl.BlockSpec(aux0.shape, lambda i: (0, 0)),
            pl.BlockSpec(w0t.shape, lambda i: (0, 0)),
            pl.BlockSpec((1, C, HW), lambda i: (i, 0, 0)),
        ],
        out_specs=[
            pl.BlockSpec((1, CPAD, HW), lambda i: (i, 0, 0)),
            pl.BlockSpec((2 * CPAD, 128), lambda i: (0, 0)),
        ],
        out_shape=[
            jax.ShapeDtypeStruct((N, CPAD, HW), jnp.float32),
            jax.ShapeDtypeStruct((2 * CPAD, 128), jnp.float32),
        ],
        scratch_shapes=[
            pltpu.VMEM((C, HW + 2 * PAD), jnp.float32),
            pltpu.VMEM((KPAD, HW), jnp.float32),
        ],
        compiler_params=pltpu.CompilerParams(
            dimension_semantics=("arbitrary",),
            vmem_limit_bytes=_VMEM_LIMIT),
    )(aux0, w0t, x3)
    return h, acc


def _conv1_kernel(aux_ref, w_ref, h_ref, x_ref, out_ref, apad_ref, patch_ref,
                  *, W, fh):
    i = pl.program_id(0)

    @pl.when(i == 0)
    def _():
        apad_ref[...] = jnp.zeros_like(apad_ref)
        patch_ref[...] = jnp.zeros_like(patch_ref)

    C = x_ref.shape[1]
    fout = out_ref.shape[1]
    a = _leaky_bn(h_ref[0, 0:fh, :], aux_ref, fh)
    _fill_patch_t(a, apad_ref, patch_ref, W)
    patch_ref[9 * fh:9 * fh + C, :] = x_ref[0]
    out = jnp.dot(w_ref[...], patch_ref[...],
                  preferred_element_type=jnp.float32)
    out = out + aux_ref[:, 2:3]
    out_ref[0] = out[0:fout, :]


def _conv1(h, x3, aux1, w1t, fout, W):
    N, C, HW = x3.shape
    fh = C
    return pl.pallas_call(
        functools.partial(_conv1_kernel, W=W, fh=fh),
        grid=(N,),
        in_specs=[
            pl.BlockSpec(aux1.shape, lambda i: (0, 0)),
            pl.BlockSpec(w1t.shape, lambda i: (0, 0)),
            pl.BlockSpec((1, CPAD, HW), lambda i: (i, 0, 0)),
            pl.BlockSpec((1, C, HW), lambda i: (i, 0, 0)),
        ],
        out_specs=pl.BlockSpec((1, fout, HW), lambda i: (i, 0, 0)),
        out_shape=jax.ShapeDtypeStruct((N, fout, HW), jnp.float32),
        scratch_shapes=[
            pltpu.VMEM((fh, HW + 2 * PAD), jnp.float32),
            pltpu.VMEM((KPAD, HW), jnp.float32),
        ],
        compiler_params=pltpu.CompilerParams(
            dimension_semantics=("arbitrary",),
            vmem_limit_bytes=_VMEM_LIMIT),
    )(aux1, w1t, h, x3)


def kernel(x_nchw, w0, b0, w1, b1, bn0_g, bn0_b, bn1_g, bn1_b):
    N, C, H, W = x_nchw.shape
    HW = H * W
    fout = b1.shape[1]
    x3 = x_nchw.reshape(N, C, HW)

    w0t = jnp.transpose(w0)[0:CPAD, :]
    w1t = jnp.transpose(w1)[0:CPAD, :]

    mean0, inv_std0 = _channel_stats(x3)
    aux0 = _make_aux(mean0, inv_std0, bn0_g, bn0_b, b0)

    h, acc1 = _conv0(x3, aux0, w0t, W)

    count = float(N * HW)
    s1 = jnp.sum(acc1[0:C, :], axis=1)
    ss1 = jnp.sum(acc1[CPAD:CPAD + C, :], axis=1)
    mean1 = s1 / count
    var1 = jnp.maximum(ss1 / count - mean1 * mean1, 0.0)
    inv_std1 = 1.0 / jnp.sqrt(var1 + BN_EPS)
    aux1 = _make_aux(mean1, inv_std1, bn1_g, bn1_b, b1)

    out = _conv1(h, x3, aux1, w1t, fout, W)
    return out.reshape(N, fout, H, W)


# speedup vs baseline: 6.8393x; 1.1373x over previous
import functools

import jax
import jax.numpy as jnp
from jax import lax
from jax.experimental import pallas as pl
from jax.experimental.pallas import tpu as pltpu

BN_EPS = 1e-5
LEAKY_SLOPE = 0.3
CPAD = 16
PAD = 128
_VMEM_LIMIT = 48 * 1024 * 1024


def _tree_sum(chunks):
    while len(chunks) > 1:
        nxt = [chunks[i] + chunks[i + 1] for i in range(0, len(chunks) - 1, 2)]
        if len(chunks) % 2:
            nxt.append(chunks[-1])
        chunks = nxt
    return chunks[0]


def _chunk_stats(v, nchunk):
    cs = [v[:, k * 128:(k + 1) * 128] for k in range(nchunk)]
    s = _tree_sum(cs)
    q = _tree_sum([c * c for c in cs])
    return s, q


def _stats_kernel(x_ref, acc_ref, *, nchunk):
    i = pl.program_id(0)

    @pl.when(i == 0)
    def _():
        acc_ref[...] = jnp.zeros_like(acc_ref)

    B, C, HW = x_ref.shape
    x = _tree_sum([x_ref[b] for b in range(B)])
    xx = _tree_sum([x_ref[b] * x_ref[b] for b in range(B)])
    s = _tree_sum([x[:, k * 128:(k + 1) * 128] for k in range(nchunk)])
    q = _tree_sum([xx[:, k * 128:(k + 1) * 128] for k in range(nchunk)])
    acc_ref[0:C, :] += s
    acc_ref[CPAD:CPAD + C, :] += q


def _channel_stats(x3):
    N, C, HW = x3.shape
    B = 8 if N % 8 == 0 else 1
    acc = pl.pallas_call(
        functools.partial(_stats_kernel, nchunk=HW // 128),
        grid=(N // B,),
        in_specs=[pl.BlockSpec((B, C, HW), lambda i: (i, 0, 0))],
        out_specs=pl.BlockSpec((2 * CPAD, 128), lambda i: (0, 0)),
        out_shape=jax.ShapeDtypeStruct((2 * CPAD, 128), jnp.float32),
        compiler_params=pltpu.CompilerParams(
            dimension_semantics=("arbitrary",),
            vmem_limit_bytes=_VMEM_LIMIT),
    )(x3)
    count = float(N * HW)
    s = jnp.sum(acc[0:C, :], axis=1)
    ss = jnp.sum(acc[CPAD:CPAD + C, :], axis=1)
    mean = s / count
    var = jnp.maximum(ss / count - mean * mean, 0.0)
    inv_std = 1.0 / jnp.sqrt(var + BN_EPS)
    return mean, inv_std


def _make_aux(mean, inv_std, gamma, beta, bias):
    c = mean.shape[0]
    fo = bias.shape[1]
    scale = gamma.reshape(c) * inv_std
    shift = beta.reshape(c) - mean * scale
    aux = jnp.zeros((CPAD, 8), jnp.float32)
    aux = aux.at[:c, 0].set(scale).at[:c, 1].set(shift)
    return aux.at[:fo, 2].set(bias.reshape(fo))


def _retile_taps(wt, c):
    taps = wt[:, :9 * c].reshape(CPAD, 9, c)
    taps = jnp.pad(taps, ((0, 0), (0, 0), (0, CPAD - c)))
    return taps.reshape(CPAD, 9 * CPAD)


def _fill_patch_t(a_bf, apad_ref, patch_ref, W, HW):
    apad_ref[0:a_bf.shape[0], PAD:PAD + HW] = a_bf
    col = lax.broadcasted_iota(jnp.int32, (CPAD, HW), 1) % W
    mask_l = col > 0
    mask_r = col < (W - 1)
    t = 0
    for dy in range(3):
        for dx in range(3):
            off = (dy - 1) * W + (dx - 1)
            src = apad_ref[:, PAD + off:PAD + off + HW]
            if dx == 0:
                src = jnp.where(mask_l, src, jnp.zeros_like(src))
            elif dx == 2:
                src = jnp.where(mask_r, src, jnp.zeros_like(src))
            patch_ref[t * CPAD:(t + 1) * CPAD, :] = src
            t += 1


def _leaky_bn(x, aux_ref, c):
    scale = aux_ref[0:c, 0:1]
    shift = aux_ref[0:c, 1:2]
    a = x * scale + shift
    return jnp.maximum(a, LEAKY_SLOPE * a)


def _conv0_kernel(aux_ref, w_ref, x_ref, h_ref, acc_ref, apad_ref, patch_ref,
                  *, W, nchunk):
    i = pl.program_id(0)

    @pl.when(i == 0)
    def _():
        acc_ref[...] = jnp.zeros_like(acc_ref)
        apad_ref[...] = jnp.zeros_like(apad_ref)

    _, C, HW = x_ref.shape
    a = _leaky_bn(x_ref[0], aux_ref, C)
    _fill_patch_t(a.astype(jnp.bfloat16), apad_ref, patch_ref, W, HW)
    h = jnp.dot(w_ref[...], patch_ref[...],
                preferred_element_type=jnp.float32)
    h = h + aux_ref[:, 2:3]
    h_ref[0] = h.astype(jnp.bfloat16)
    s, q = _chunk_stats(h, nchunk)
    acc_ref[0:CPAD, :] += s
    acc_ref[CPAD:, :] += q


def _conv0(x3, aux0, w0b, W):
    N, C, HW = x3.shape
    h, acc = pl.pallas_call(
        functools.partial(_conv0_kernel, W=W, nchunk=HW // 128),
        grid=(N,),
        in_specs=[
            pl.BlockSpec(aux0.shape, lambda i: (0, 0)),
            pl.BlockSpec(w0b.shape, lambda i: (0, 0)),
            pl.BlockSpec((1, C, HW), lambda i: (i, 0, 0)),
        ],
        out_specs=[
            pl.BlockSpec((1, CPAD, HW), lambda i: (i, 0, 0)),
            pl.BlockSpec((2 * CPAD, 128), lambda i: (0, 0)),
        ],
        out_shape=[
            jax.ShapeDtypeStruct((N, CPAD, HW), jnp.bfloat16),
            jax.ShapeDtypeStruct((2 * CPAD, 128), jnp.float32),
        ],
        scratch_shapes=[
            pltpu.VMEM((CPAD, HW + 2 * PAD), jnp.bfloat16),
            pltpu.VMEM((9 * CPAD, HW), jnp.bfloat16),
        ],
        compiler_params=pltpu.CompilerParams(
            dimension_semantics=("arbitrary",),
            vmem_limit_bytes=_VMEM_LIMIT),
    )(aux0, w0b, x3)
    return h, acc


def _conv1_kernel(aux_ref, w_ref, h_ref, x_ref, out_ref, apad_ref, patch_ref,
                  *, W, fh):
    i = pl.program_id(0)

    @pl.when(i == 0)
    def _():
        apad_ref[...] = jnp.zeros_like(apad_ref)

    _, C, HW = x_ref.shape
    fout = out_ref.shape[1]
    a = _leaky_bn(h_ref[0, 0:fh, :].astype(jnp.float32), aux_ref, fh)
    _fill_patch_t(a.astype(jnp.bfloat16), apad_ref, patch_ref, W, HW)
    patch_ref[9 * CPAD:9 * CPAD + C, :] = x_ref[0].astype(jnp.bfloat16)
    out = jnp.dot(w_ref[...], patch_ref[...],
                  preferred_element_type=jnp.float32)
    out = out + aux_ref[:, 2:3]
    out_ref[0] = out[0:fout, :]


def _conv1(h, x3, aux1, w1b, fout, W):
    N, C, HW = x3.shape
    fh = C
    return pl.pallas_call(
        functools.partial(_conv1_kernel, W=W, fh=fh),
        grid=(N,),
        in_specs=[
            pl.BlockSpec(aux1.shape, lambda i: (0, 0)),
            pl.BlockSpec(w1b.shape, lambda i: (0, 0)),
            pl.BlockSpec((1, CPAD, HW), lambda i: (i, 0, 0)),
            pl.BlockSpec((1, C, HW), lambda i: (i, 0, 0)),
        ],
        out_specs=pl.BlockSpec((1, fout, HW), lambda i: (i, 0, 0)),
        out_shape=jax.ShapeDtypeStruct((N, fout, HW), jnp.float32),
        scratch_shapes=[
            pltpu.VMEM((CPAD, HW + 2 * PAD), jnp.bfloat16),
            pltpu.VMEM((9 * CPAD + C, HW), jnp.bfloat16),
        ],
        compiler_params=pltpu.CompilerParams(
            dimension_semantics=("arbitrary",),
            vmem_limit_bytes=_VMEM_LIMIT),
    )(aux1, w1b, h, x3)


def kernel(x_nchw, w0, b0, w1, b1, bn0_g, bn0_b, bn1_g, bn1_b):
    N, C, H, W = x_nchw.shape
    HW = H * W
    fout = b1.shape[1]
    x3 = x_nchw.reshape(N, C, HW)

    w0t = jnp.transpose(w0)[0:CPAD, :]
    w1t = jnp.transpose(w1)[0:CPAD, :]
    w0b = _retile_taps(w0t, C).astype(jnp.bfloat16)
    w1b = jnp.concatenate(
        [_retile_taps(w1t, C), w1t[:, 9 * C:9 * C + C]],
        axis=1).astype(jnp.bfloat16)

    mean0, inv_std0 = _channel_stats(x3)
    aux0 = _make_aux(mean0, inv_std0, bn0_g, bn0_b, b0)

    h, acc1 = _conv0(x3, aux0, w0b, W)

    count = float(N * HW)
    s1 = jnp.sum(acc1[0:C, :], axis=1)
    ss1 = jnp.sum(acc1[CPAD:CPAD + C, :], axis=1)
    mean1 = s1 / count
    var1 = jnp.maximum(ss1 / count - mean1 * mean1, 0.0)
    inv_std1 = 1.0 / jnp.sqrt(var1 + BN_EPS)
    aux1 = _make_aux(mean1, inv_std1, bn1_g, bn1_b, b1)

    out = _conv1(h, x3, aux1, w1b, fout, W)
    return out.reshape(N, fout, H, W)


# speedup vs baseline: 9.3891x; 1.3728x over previous
import functools

import jax
import jax.numpy as jnp
from jax import lax
from jax.experimental import pallas as pl
from jax.experimental.pallas import tpu as pltpu

BN_EPS = 1e-5
LEAKY_SLOPE = 0.3
CPAD = 16
PAD = 128
_VMEM_LIMIT = 48 * 1024 * 1024


def _tree_sum(chunks):
    while len(chunks) > 1:
        nxt = [chunks[i] + chunks[i + 1] for i in range(0, len(chunks) - 1, 2)]
        if len(chunks) % 2:
            nxt.append(chunks[-1])
        chunks = nxt
    return chunks[0]


def _chunk_stats(v, nchunk):
    cs = [v[:, k * 128:(k + 1) * 128] for k in range(nchunk)]
    s = _tree_sum(cs)
    q = _tree_sum([c * c for c in cs])
    return s, q


def _stats_kernel(x_ref, acc_ref, *, nchunk):
    i = pl.program_id(0)

    @pl.when(i == 0)
    def _():
        acc_ref[...] = jnp.zeros_like(acc_ref)

    B, C, HW = x_ref.shape
    x = _tree_sum([x_ref[b] for b in range(B)])
    xx = _tree_sum([x_ref[b] * x_ref[b] for b in range(B)])
    s = _tree_sum([x[:, k * 128:(k + 1) * 128] for k in range(nchunk)])
    q = _tree_sum([xx[:, k * 128:(k + 1) * 128] for k in range(nchunk)])
    acc_ref[0:C, :] += s
    acc_ref[CPAD:CPAD + C, :] += q


def _channel_stats(x3):
    N, C, HW = x3.shape
    B = 8 if N % 8 == 0 else 1
    acc = pl.pallas_call(
        functools.partial(_stats_kernel, nchunk=HW // 128),
        grid=(N // B,),
        in_specs=[pl.BlockSpec((B, C, HW), lambda i: (i, 0, 0))],
        out_specs=pl.BlockSpec((2 * CPAD, 128), lambda i: (0, 0)),
        out_shape=jax.ShapeDtypeStruct((2 * CPAD, 128), jnp.float32),
        compiler_params=pltpu.CompilerParams(
            dimension_semantics=("arbitrary",),
            vmem_limit_bytes=_VMEM_LIMIT),
    )(x3)
    count = float(N * HW)
    s = jnp.sum(acc[0:C, :], axis=1)
    ss = jnp.sum(acc[CPAD:CPAD + C, :], axis=1)
    mean = s / count
    var = jnp.maximum(ss / count - mean * mean, 0.0)
    inv_std = 1.0 / jnp.sqrt(var + BN_EPS)
    return mean, inv_std


def _make_aux(mean, inv_std, gamma, beta, bias):
    c = mean.shape[0]
    fo = bias.shape[1]
    scale = gamma.reshape(c) * inv_std
    shift = beta.reshape(c) - mean * scale
    aux = jnp.zeros((CPAD, 8), jnp.float32)
    aux = aux.at[:c, 0].set(scale).at[:c, 1].set(shift)
    return aux.at[:fo, 2].set(bias.reshape(fo))


def _retile_taps(wt, c):
    taps = wt[:, :9 * c].reshape(CPAD, 9, c)
    taps = jnp.pad(taps, ((0, 0), (0, 0), (0, CPAD - c)))
    return taps.reshape(CPAD, 9 * CPAD)


def _fill_patch_t(a_bf, apad_ref, patch_ref, W, HW):
    apad_ref[0:a_bf.shape[0], PAD:PAD + HW] = a_bf
    col = lax.broadcasted_iota(jnp.int32, (CPAD, HW), 1) % W
    mask_l = col > 0
    mask_r = col < (W - 1)
    t = 0
    for dy in range(3):
        for dx in range(3):
            off = (dy - 1) * W + (dx - 1)
            src = apad_ref[:, PAD + off:PAD + off + HW]
            if dx == 0:
                src = jnp.where(mask_l, src, jnp.zeros_like(src))
            elif dx == 2:
                src = jnp.where(mask_r, src, jnp.zeros_like(src))
            patch_ref[t * CPAD:(t + 1) * CPAD, :] = src
            t += 1


def _leaky_bn(x, aux_ref, c):
    scale = aux_ref[0:c, 0:1]
    shift = aux_ref[0:c, 1:2]
    a = x * scale + shift
    return jnp.maximum(a, LEAKY_SLOPE * a)


IMGB = 4


def _conv0_kernel(aux_ref, w_ref, x_ref, h_ref, acc_ref, apad_ref, patch_ref,
                  *, W, nchunk):
    i = pl.program_id(0)

    @pl.when(i == 0)
    def _():
        acc_ref[...] = jnp.zeros_like(acc_ref)
        apad_ref[...] = jnp.zeros_like(apad_ref)

    B, C, HW = x_ref.shape
    for b in range(B):
        a = _leaky_bn(x_ref[b], aux_ref, C)
        _fill_patch_t(a.astype(jnp.bfloat16), apad_ref.at[b],
                      patch_ref.at[:, b * HW:(b + 1) * HW], W, HW)
    h = jnp.dot(w_ref[...], patch_ref[...],
                preferred_element_type=jnp.float32)
    h = h + aux_ref[:, 2:3]
    for b in range(B):
        h_ref[b] = h[:, b * HW:(b + 1) * HW].astype(jnp.bfloat16)
    s, q = _chunk_stats(h, B * nchunk)
    acc_ref[0:CPAD, :] += s
    acc_ref[CPAD:, :] += q


def _conv0(x3, aux0, w0b, W):
    N, C, HW = x3.shape
    B = IMGB if N % IMGB == 0 else 1
    h, acc = pl.pallas_call(
        functools.partial(_conv0_kernel, W=W, nchunk=HW // 128),
        grid=(N // B,),
        in_specs=[
            pl.BlockSpec(aux0.shape, lambda i: (0, 0)),
            pl.BlockSpec(w0b.shape, lambda i: (0, 0)),
            pl.BlockSpec((B, C, HW), lambda i: (i, 0, 0)),
        ],
        out_specs=[
            pl.BlockSpec((B, CPAD, HW), lambda i: (i, 0, 0)),
            pl.BlockSpec((2 * CPAD, 128), lambda i: (0, 0)),
        ],
        out_shape=[
            jax.ShapeDtypeStruct((N, CPAD, HW), jnp.bfloat16),
            jax.ShapeDtypeStruct((2 * CPAD, 128), jnp.float32),
        ],
        scratch_shapes=[
            pltpu.VMEM((B, CPAD, HW + 2 * PAD), jnp.bfloat16),
            pltpu.VMEM((9 * CPAD, B * HW), jnp.bfloat16),
        ],
        compiler_params=pltpu.CompilerParams(
            dimension_semantics=("arbitrary",),
            vmem_limit_bytes=_VMEM_LIMIT),
    )(aux0, w0b, x3)
    return h, acc


def _conv1_kernel(aux_ref, w_ref, h_ref, x_ref, out_ref, apad_ref, patch_ref,
                  *, W, fh):
    i = pl.program_id(0)

    @pl.when(i == 0)
    def _():
        apad_ref[...] = jnp.zeros_like(apad_ref)

    B, C, HW = x_ref.shape
    fout = out_ref.shape[1]
    for b in range(B):
        a = _leaky_bn(h_ref[b, 0:fh, :].astype(jnp.float32), aux_ref, fh)
        pv = patch_ref.at[:, b * HW:(b + 1) * HW]
        _fill_patch_t(a.astype(jnp.bfloat16), apad_ref.at[b], pv, W, HW)
        pv[9 * CPAD:9 * CPAD + C, :] = x_ref[b].astype(jnp.bfloat16)
    out = jnp.dot(w_ref[...], patch_ref[...],
                  preferred_element_type=jnp.float32)
    out = out + aux_ref[:, 2:3]
    for b in range(B):
        out_ref[b] = out[0:fout, b * HW:(b + 1) * HW]


def _conv1(h, x3, aux1, w1b, fout, W):
    N, C, HW = x3.shape
    fh = C
    B = IMGB if N % IMGB == 0 else 1
    return pl.pallas_call(
        functools.partial(_conv1_kernel, W=W, fh=fh),
        grid=(N // B,),
        in_specs=[
            pl.BlockSpec(aux1.shape, lambda i: (0, 0)),
            pl.BlockSpec(w1b.shape, lambda i: (0, 0)),
            pl.BlockSpec((B, CPAD, HW), lambda i: (i, 0, 0)),
            pl.BlockSpec((B, C, HW), lambda i: (i, 0, 0)),
        ],
        out_specs=pl.BlockSpec((B, fout, HW), lambda i: (i, 0, 0)),
        out_shape=jax.ShapeDtypeStruct((N, fout, HW), jnp.float32),
        scratch_shapes=[
            pltpu.VMEM((B, CPAD, HW + 2 * PAD), jnp.bfloat16),
            pltpu.VMEM((9 * CPAD + C, B * HW), jnp.bfloat16),
        ],
        compiler_params=pltpu.CompilerParams(
            dimension_semantics=("arbitrary",),
            vmem_limit_bytes=_VMEM_LIMIT),
    )(aux1, w1b, h, x3)


def kernel(x_nchw, w0, b0, w1, b1, bn0_g, bn0_b, bn1_g, bn1_b):
    N, C, H, W = x_nchw.shape
    HW = H * W
    fout = b1.shape[1]
    x3 = x_nchw.reshape(N, C, HW)

    w0t = jnp.transpose(w0)[0:CPAD, :]
    w1t = jnp.transpose(w1)[0:CPAD, :]
    w0b = _retile_taps(w0t, C).astype(jnp.bfloat16)
    w1b = jnp.concatenate(
        [_retile_taps(w1t, C), w1t[:, 9 * C:9 * C + C]],
        axis=1).astype(jnp.bfloat16)

    mean0, inv_std0 = _channel_stats(x3)
    aux0 = _make_aux(mean0, inv_std0, bn0_g, bn0_b, b0)

    h, acc1 = _conv0(x3, aux0, w0b, W)

    count = float(N * HW)
    s1 = jnp.sum(acc1[0:C, :], axis=1)
    ss1 = jnp.sum(acc1[CPAD:CPAD + C, :], axis=1)
    mean1 = s1 / count
    var1 = jnp.maximum(ss1 / count - mean1 * mean1, 0.0)
    inv_std1 = 1.0 / jnp.sqrt(var1 + BN_EPS)
    aux1 = _make_aux(mean1, inv_std1, bn1_g, bn1_b, b1)

    out = _conv1(h, x3, aux1, w1b, fout, W)
    return out.reshape(N, fout, H, W)


# speedup vs baseline: 9.5589x; 1.0181x over previous
import functools

import jax
import jax.numpy as jnp
from jax import lax
from jax.experimental import pallas as pl
from jax.experimental.pallas import tpu as pltpu

BN_EPS = 1e-5
LEAKY_SLOPE = 0.3
CPAD = 16
PAD = 128
_VMEM_LIMIT = 48 * 1024 * 1024


def _tree_sum(chunks):
    while len(chunks) > 1:
        nxt = [chunks[i] + chunks[i + 1] for i in range(0, len(chunks) - 1, 2)]
        if len(chunks) % 2:
            nxt.append(chunks[-1])
        chunks = nxt
    return chunks[0]


def _chunk_stats(v, nchunk):
    cs = [v[:, k * 128:(k + 1) * 128] for k in range(nchunk)]
    s = _tree_sum(cs)
    q = _tree_sum([c * c for c in cs])
    return s, q


def _stats_kernel(x_ref, acc_ref, *, nchunk):
    i = pl.program_id(0)

    @pl.when(i == 0)
    def _():
        acc_ref[...] = jnp.zeros_like(acc_ref)

    B, C, HW = x_ref.shape
    x = _tree_sum([x_ref[b] for b in range(B)])
    xx = _tree_sum([x_ref[b] * x_ref[b] for b in range(B)])
    s = _tree_sum([x[:, k * 128:(k + 1) * 128] for k in range(nchunk)])
    q = _tree_sum([xx[:, k * 128:(k + 1) * 128] for k in range(nchunk)])
    acc_ref[0:C, :] += s
    acc_ref[CPAD:CPAD + C, :] += q


def _channel_stats(x3):
    N, C, HW = x3.shape
    B = 8 if N % 8 == 0 else 1
    acc = pl.pallas_call(
        functools.partial(_stats_kernel, nchunk=HW // 128),
        grid=(N // B,),
        in_specs=[pl.BlockSpec((B, C, HW), lambda i: (i, 0, 0))],
        out_specs=pl.BlockSpec((2 * CPAD, 128), lambda i: (0, 0)),
        out_shape=jax.ShapeDtypeStruct((2 * CPAD, 128), jnp.float32),
        compiler_params=pltpu.CompilerParams(
            dimension_semantics=("arbitrary",),
            vmem_limit_bytes=_VMEM_LIMIT),
    )(x3)
    count = float(N * HW)
    s = jnp.sum(acc[0:C, :], axis=1)
    ss = jnp.sum(acc[CPAD:CPAD + C, :], axis=1)
    mean = s / count
    var = jnp.maximum(ss / count - mean * mean, 0.0)
    inv_std = 1.0 / jnp.sqrt(var + BN_EPS)
    return mean, inv_std


def _make_aux(mean, inv_std, gamma, beta, bias):
    c = mean.shape[0]
    fo = bias.shape[1]
    scale = gamma.reshape(c) * inv_std
    shift = beta.reshape(c) - mean * scale
    aux = jnp.zeros((CPAD, 8), jnp.float32)
    aux = aux.at[:c, 0].set(scale).at[:c, 1].set(shift)
    return aux.at[:fo, 2].set(bias.reshape(fo))


def _retile_taps(wt, c):
    taps = wt[:, :9 * c].reshape(CPAD, 9, c)
    taps = jnp.pad(taps, ((0, 0), (0, 0), (0, CPAD - c)))
    return taps.reshape(CPAD, 9 * CPAD)


def _fill_patch_t(a_bf, apad_ref, patch_ref, W, HW):
    apad_ref[0:a_bf.shape[0], PAD:PAD + HW] = a_bf
    col = lax.broadcasted_iota(jnp.int32, (CPAD, HW), 1) % W
    mask_l = col > 0
    mask_r = col < (W - 1)
    t = 0
    for dy in range(3):
        for dx in range(3):
            off = (dy - 1) * W + (dx - 1)
            src = apad_ref[:, PAD + off:PAD + off + HW]
            if dx == 0:
                src = jnp.where(mask_l, src, jnp.zeros_like(src))
            elif dx == 2:
                src = jnp.where(mask_r, src, jnp.zeros_like(src))
            patch_ref[t * CPAD:(t + 1) * CPAD, :] = src
            t += 1


def _leaky_bn(x, aux_ref, c):
    scale = aux_ref[0:c, 0:1]
    shift = aux_ref[0:c, 1:2]
    a = x * scale + shift
    return jnp.maximum(a, LEAKY_SLOPE * a)


IMGB = 8


def _conv0_kernel(aux_ref, w_ref, x_ref, h_ref, acc_ref, apad_ref, patch_ref,
                  *, W, nchunk):
    i = pl.program_id(0)

    @pl.when(i == 0)
    def _():
        acc_ref[...] = jnp.zeros_like(acc_ref)
        apad_ref[...] = jnp.zeros_like(apad_ref)

    B, C, HW = x_ref.shape
    for b in range(B):
        a = _leaky_bn(x_ref[b], aux_ref, C)
        _fill_patch_t(a.astype(jnp.bfloat16), apad_ref.at[b],
                      patch_ref.at[:, b * HW:(b + 1) * HW], W, HW)
    h = jnp.dot(w_ref[...], patch_ref[...],
                preferred_element_type=jnp.float32)
    h = h + aux_ref[:, 2:3]
    for b in range(B):
        h_ref[b] = h[:, b * HW:(b + 1) * HW].astype(jnp.bfloat16)
    s, q = _chunk_stats(h, B * nchunk)
    acc_ref[0:CPAD, :] += s
    acc_ref[CPAD:, :] += q


def _conv0(x3, aux0, w0b, W):
    N, C, HW = x3.shape
    B = IMGB if N % IMGB == 0 else 1
    h, acc = pl.pallas_call(
        functools.partial(_conv0_kernel, W=W, nchunk=HW // 128),
        grid=(N // B,),
        in_specs=[
            pl.BlockSpec(aux0.shape, lambda i: (0, 0)),
            pl.BlockSpec(w0b.shape, lambda i: (0, 0)),
            pl.BlockSpec((B, C, HW), lambda i: (i, 0, 0)),
        ],
        out_specs=[
            pl.BlockSpec((B, CPAD, HW), lambda i: (i, 0, 0)),
            pl.BlockSpec((2 * CPAD, 128), lambda i: (0, 0)),
        ],
        out_shape=[
            jax.ShapeDtypeStruct((N, CPAD, HW), jnp.bfloat16),
            jax.ShapeDtypeStruct((2 * CPAD, 128), jnp.float32),
        ],
        scratch_shapes=[
            pltpu.VMEM((B, CPAD, HW + 2 * PAD), jnp.bfloat16),
            pltpu.VMEM((9 * CPAD, B * HW), jnp.bfloat16),
        ],
        compiler_params=pltpu.CompilerParams(
            dimension_semantics=("arbitrary",),
            vmem_limit_bytes=_VMEM_LIMIT),
    )(aux0, w0b, x3)
    return h, acc


def _conv1_kernel(aux_ref, w_ref, h_ref, x_ref, out_ref, apad_ref, patch_ref,
                  *, W, fh):
    i = pl.program_id(0)

    @pl.when(i == 0)
    def _():
        apad_ref[...] = jnp.zeros_like(apad_ref)

    B, C, HW = x_ref.shape
    fout = out_ref.shape[1]
    for b in range(B):
        a = _leaky_bn(h_ref[b, 0:fh, :].astype(jnp.float32), aux_ref, fh)
        pv = patch_ref.at[:, b * HW:(b + 1) * HW]
        _fill_patch_t(a.astype(jnp.bfloat16), apad_ref.at[b], pv, W, HW)
        pv[9 * CPAD:9 * CPAD + C, :] = x_ref[b].astype(jnp.bfloat16)
    out = jnp.dot(w_ref[...], patch_ref[...],
                  preferred_element_type=jnp.float32)
    out = out + aux_ref[:, 2:3]
    for b in range(B):
        out_ref[b] = out[0:fout, b * HW:(b + 1) * HW]


def _conv1(h, x3, aux1, w1b, fout, W):
    N, C, HW = x3.shape
    fh = C
    B = IMGB if N % IMGB == 0 else 1
    return pl.pallas_call(
        functools.partial(_conv1_kernel, W=W, fh=fh),
        grid=(N // B,),
        in_specs=[
            pl.BlockSpec(aux1.shape, lambda i: (0, 0)),
            pl.BlockSpec(w1b.shape, lambda i: (0, 0)),
            pl.BlockSpec((B, CPAD, HW), lambda i: (i, 0, 0)),
            pl.BlockSpec((B, C, HW), lambda i: (i, 0, 0)),
        ],
        out_specs=pl.BlockSpec((B, fout, HW), lambda i: (i, 0, 0)),
        out_shape=jax.ShapeDtypeStruct((N, fout, HW), jnp.float32),
        scratch_shapes=[
            pltpu.VMEM((B, CPAD, HW + 2 * PAD), jnp.bfloat16),
            pltpu.VMEM((9 * CPAD + C, B * HW), jnp.bfloat16),
        ],
        compiler_params=pltpu.CompilerParams(
            dimension_semantics=("arbitrary",),
            vmem_limit_bytes=_VMEM_LIMIT),
    )(aux1, w1b, h, x3)


def kernel(x_nchw, w0, b0, w1, b1, bn0_g, bn0_b, bn1_g, bn1_b):
    N, C, H, W = x_nchw.shape
    HW = H * W
    fout = b1.shape[1]
    x3 = x_nchw.reshape(N, C, HW)

    w0t = jnp.transpose(w0)[0:CPAD, :]
    w1t = jnp.transpose(w1)[0:CPAD, :]
    w0b = _retile_taps(w0t, C).astype(jnp.bfloat16)
    w1b = jnp.concatenate(
        [_retile_taps(w1t, C), w1t[:, 9 * C:9 * C + C]],
        axis=1).astype(jnp.bfloat16)

    mean0, inv_std0 = _channel_stats(x3)
    aux0 = _make_aux(mean0, inv_std0, bn0_g, bn0_b, b0)

    h, acc1 = _conv0(x3, aux0, w0b, W)

    count = float(N * HW)
    s1 = jnp.sum(acc1[0:C, :], axis=1)
    ss1 = jnp.sum(acc1[CPAD:CPAD + C, :], axis=1)
    mean1 = s1 / count
    var1 = jnp.maximum(ss1 / count - mean1 * mean1, 0.0)
    inv_std1 = 1.0 / jnp.sqrt(var1 + BN_EPS)
    aux1 = _make_aux(mean1, inv_std1, bn1_g, bn1_b, b1)

    out = _conv1(h, x3, aux1, w1b, fout, W)
    return out.reshape(N, fout, H, W)
